# Initial kernel scaffold; baseline (speedup 1.0000x reference)
import jax
import jax.numpy as jnp
from jax.experimental import pallas as pl


def kernel(x, hidden, gru_W_ih_l0, gru_W_hh_l0, gru_b_ih_l0, gru_b_hh_l0, gru_W_ih_l1, gru_W_hh_l1, gru_b_ih_l1, gru_b_hh_l1, hg_W0, hg_b0, hg_W1, hg_b1, hg_W2, hg_b2, ce_W0, ce_b0, ce_W1, ce_b1, hp_W0, hp_b0, hp_W1, hp_b1, num_samples):
    raise NotImplementedError("write your pallas kernel here")


# speedup vs baseline: 1.0067x; 1.0067x over previous
import jax
import jax.numpy as jnp
from jax import lax
from jax.experimental import pallas as pl

INPUT_DIM = 2
HIDDEN = 256
T = 60
G = 64
GG = G * G
GR0 = -50.0
GR1 = 50.0
CELL = (GR1 - GR0) / G
B = 64
NS = 6
TB = 2


def _heads_body(x_ref, h0_ref, h1_ref,
                wih0_ref, whh0_ref, bih0_ref, bhh0_ref,
                wih1_ref, whh1_ref, bih1_ref, bhh1_ref,
                hgW0_ref, hgb0_ref, hgW1_ref, hgb1_ref, hgW2_ref, hgb2_ref,
                ceW0_ref, ceb0_ref, ceW1_ref, ceb1_ref,
                lh_ref, traj_ref, mc_ref):
    H = HIDDEN

    def gru(inp, h, WihT, WhhT, bih, bhh):
        gi = jnp.dot(inp, WihT) + bih
        gh = jnp.dot(h, WhhT) + bhh
        ir, iz, inn = gi[:, :H], gi[:, H:2 * H], gi[:, 2 * H:]
        hr, hz, hn = gh[:, :H], gh[:, H:2 * H], gh[:, 2 * H:]
        r = jax.nn.sigmoid(ir + hr)
        z = jax.nn.sigmoid(iz + hz)
        n = jnp.tanh(inn + r * hn)
        return (1.0 - z) * n + z * h

    h0 = gru(x_ref[...], h0_ref[...], wih0_ref[...], whh0_ref[...],
             bih0_ref[...], bhh0_ref[...])
    lh = gru(h0, h1_ref[...], wih1_ref[...], whh1_ref[...],
             bih1_ref[...], bhh1_ref[...])
    lh_ref[...] = lh

    t1 = jnp.maximum(jnp.dot(lh, hgW0_ref[...]) + hgb0_ref[...], 0.0)
    t2 = jnp.maximum(jnp.dot(t1, hgW1_ref[...]) + hgb1_ref[...], 0.0)
    traj_ref[...] = jnp.dot(t2, hgW2_ref[...]) + hgb2_ref[...]

    c1 = jnp.maximum(jnp.dot(lh, ceW0_ref[...]) + ceb0_ref[...], 0.0)
    conf = jnp.dot(c1, ceW1_ref[...]) + ceb1_ref[...]
    mc_ref[...] = jnp.mean(conf, axis=1, keepdims=True)


def _heat_body(lh_ref, snz_ref, w0_ref, b0_ref, w1_ref, b1_ref, g_ref,
               xc_ref, yc_ref):
    lh = lh_ref[...]
    th = (lh[None, :, :] + snz_ref[...]).reshape(TB * B, HIDDEN)
    hpre = jnp.maximum(jnp.dot(th, w0_ref[...]) + b0_ref[...], 0.0)
    hm = jnp.dot(hpre, w1_ref[...]) + b1_ref[...]
    m = jnp.max(hm, axis=-1, keepdims=True)
    e = jnp.exp(hm - m)
    heat = e / jnp.sum(e, axis=-1, keepdims=True)
    logp = jnp.log(jnp.clip(heat, 1e-30, 1.0))
    iota = lax.broadcasted_iota(jnp.int32, (TB * B, GG), 1)
    for s in range(NS - 1):
        v = g_ref[s, 0].reshape(TB * B, GG) + logp
        vm = jnp.max(v, axis=-1, keepdims=True)
        idx = jnp.min(jnp.where(v == vm, iota, GG), axis=-1)
        xc = GR0 + (idx % G).astype(jnp.float32) * CELL + CELL / 2.0
        yc = GR0 + (idx // G).astype(jnp.float32) * CELL + CELL / 2.0
        xc_ref[s, 0] = xc.reshape(TB, B)
        yc_ref[s, 0] = yc.reshape(TB, B)


def kernel(x, hidden, gru_W_ih_l0, gru_W_hh_l0, gru_b_ih_l0, gru_b_hh_l0,
           gru_W_ih_l1, gru_W_hh_l1, gru_b_ih_l1, gru_b_hh_l1,
           hg_W0, hg_b0, hg_W1, hg_b1, hg_W2, hg_b2,
           ce_W0, ce_b0, ce_W1, ce_b1,
           hp_W0, hp_b0, hp_W1, hp_b1, num_samples):
    f32 = jnp.float32
    x2 = x[:, 0, :]
    row = lambda b: b.reshape(1, -1)

    lh, traj, mc = pl.pallas_call(
        _heads_body,
        out_shape=(
            jax.ShapeDtypeStruct((B, HIDDEN), f32),
            jax.ShapeDtypeStruct((B, 2 * T), f32),
            jax.ShapeDtypeStruct((B, 1), f32),
        ),
    )(x2, hidden[0], hidden[1],
      gru_W_ih_l0.T, gru_W_hh_l0.T, row(gru_b_ih_l0), row(gru_b_hh_l0),
      gru_W_ih_l1.T, gru_W_hh_l1.T, row(gru_b_ih_l1), row(gru_b_hh_l1),
      hg_W0.T, row(hg_b0), hg_W1.T, row(hg_b1), hg_W2.T, row(hg_b2),
      ce_W0.T, row(ce_b0), ce_W1.T, row(ce_b1))

    noise = jax.random.normal(jax.random.key(42), (T, B, HIDDEN), dtype=f32)
    scale = 0.1 * (jnp.arange(T, dtype=f32) / T)[:, None, None]
    snoise = noise * scale
    gmb = jax.random.gumbel(jax.random.key(7), (NS - 1, T, B, GG), f32)

    NT = T // TB
    xc, yc = pl.pallas_call(
        _heat_body,
        grid=(NT,),
        in_specs=[
            pl.BlockSpec((B, HIDDEN), lambda i: (0, 0)),
            pl.BlockSpec((TB, B, HIDDEN), lambda i: (i, 0, 0)),
            pl.BlockSpec((HIDDEN, HIDDEN), lambda i: (0, 0)),
            pl.BlockSpec((1, HIDDEN), lambda i: (0, 0)),
            pl.BlockSpec((HIDDEN, GG), lambda i: (0, 0)),
            pl.BlockSpec((1, GG), lambda i: (0, 0)),
            pl.BlockSpec((NS - 1, 1, TB, B, GG), lambda i: (0, i, 0, 0, 0)),
        ],
        out_specs=[
            pl.BlockSpec((NS - 1, 1, TB, B), lambda i: (0, i, 0, 0)),
            pl.BlockSpec((NS - 1, 1, TB, B), lambda i: (0, i, 0, 0)),
        ],
        out_shape=(
            jax.ShapeDtypeStruct((NS - 1, NT, TB, B), f32),
            jax.ShapeDtypeStruct((NS - 1, NT, TB, B), f32),
        ),
    )(lh, snoise, hp_W0.T, row(hp_b0), hp_W1.T, row(hp_b1),
      gmb.reshape(NS - 1, NT, TB, B, GG))

    xc = xc.reshape(NS - 1, T, B)
    yc = yc.reshape(NS - 1, T, B)
    samp = jnp.stack([xc, yc], axis=-1)
    samp = jnp.transpose(samp, (2, 0, 1, 3))
    traj = traj.reshape(B, T, 2)
    preds = jnp.concatenate([traj[:, None, :, :], samp], axis=1)
    ns_f = jnp.asarray(num_samples, dtype=f32)
    decay = 0.9 ** (jnp.arange(NS, dtype=f32) % ns_f)
    confs = mc * decay[None, :]
    return preds, confs
